# baseline (speedup 1.0000x reference)
import jax
import jax.numpy as jnp
from jax import lax
from jax.experimental import pallas as pl
from jax.experimental.pallas import tpu as pltpu
from jax.experimental.pallas import tpu_sc as plsc

DIM = 64
B_TOTAL = 16384
NC = 2
NS = 16
L = 16
NW = NC * NS
BPW = B_TOTAL // NW
NCHUNK = 4
CHUNK = BPW // NCHUNK
NGROUP = BPW // L
REG = 0.0001


def _sc_body(uids_hbm, pos_hbm, neg_hbm, uemb_hbm, iemb_hbm,
             diff_hbm, sq_hbm,
             idx_u, idx_p, idx_n, u_v, p_v, n_v, d_v, sq_v, sem):
    wid = lax.axis_index("s") * NC + lax.axis_index("c")
    pltpu.sync_copy(uids_hbm.at[wid], idx_u)
    pltpu.sync_copy(pos_hbm.at[wid], idx_p)
    pltpu.sync_copy(neg_hbm.at[wid], idx_n)
    cps = []
    for j in range(NCHUNK):
        dst = pl.ds(j * CHUNK, CHUNK)
        cps.append(pltpu.async_copy(uemb_hbm.at[idx_u.at[j]], u_v.at[dst], sem))
        cps.append(pltpu.async_copy(iemb_hbm.at[idx_p.at[j]], p_v.at[dst], sem))
        cps.append(pltpu.async_copy(iemb_hbm.at[idx_n.at[j]], n_v.at[dst], sem))
    for c in cps:
        c.wait()

    lane = lax.iota(jnp.int32, L)

    def group(g, acc_sq):
        rows = g * L + lane
        acc_d = jnp.zeros((L,), jnp.float32)
        for k in range(DIM):
            cols = jnp.full((L,), k, jnp.int32)
            u = plsc.load_gather(u_v, [rows, cols])
            p = plsc.load_gather(p_v, [rows, cols])
            n = plsc.load_gather(n_v, [rows, cols])
            acc_d = acc_d + u * (p - n)
            acc_sq = acc_sq + (u * u + (p * p + n * n))
        d_v[pl.ds(g * L, L)] = acc_d
        return acc_sq

    acc_sq = lax.fori_loop(0, NGROUP, group, jnp.zeros((L,), jnp.float32))
    sq_v[...] = acc_sq
    pltpu.sync_copy(d_v, diff_hbm.at[pl.ds(wid * BPW, BPW)])
    pltpu.sync_copy(sq_v, sq_hbm.at[pl.ds(wid * L, L)])


def _tc_body(diff_ref, sq_ref, total_ref, bpr_ref, reg_ref):
    d = diff_ref[...]
    bpr = -jnp.sum(jnp.log(jax.nn.sigmoid(d))) / B_TOTAL
    reg = REG * (jnp.sum(sq_ref[...]) / B_TOTAL)
    total_ref[...] = jnp.reshape(bpr + reg, (1, 1))
    bpr_ref[...] = jnp.reshape(bpr, (1, 1))
    reg_ref[...] = jnp.reshape(reg, (1, 1))


def _build_sc(interpret=False):
    mesh = plsc.VectorSubcoreMesh(
        core_axis_name="c", subcore_axis_name="s",
        num_cores=NC, num_subcores=NS)
    return pl.kernel(
        _sc_body,
        out_type=[
            jax.ShapeDtypeStruct((B_TOTAL,), jnp.float32),
            jax.ShapeDtypeStruct((NW * L,), jnp.float32),
        ],
        mesh=mesh,
        compiler_params=pltpu.CompilerParams(needs_layout_passes=False, use_tc_tiling_on_sc=False),
        scratch_types=[
            pltpu.VMEM((NCHUNK, CHUNK), jnp.int32),
            pltpu.VMEM((NCHUNK, CHUNK), jnp.int32),
            pltpu.VMEM((NCHUNK, CHUNK), jnp.int32),
            pltpu.VMEM((BPW, DIM), jnp.float32),
            pltpu.VMEM((BPW, DIM), jnp.float32),
            pltpu.VMEM((BPW, DIM), jnp.float32),
            pltpu.VMEM((BPW,), jnp.float32),
            pltpu.VMEM((L,), jnp.float32),
            pltpu.SemaphoreType.DMA,
        ],
        interpret=interpret,
    )


def kernel(uids, pos, neg, user_emb, item_emb):
    u3 = uids.reshape(NW, NCHUNK, CHUNK)
    p3 = pos.reshape(NW, NCHUNK, CHUNK)
    n3 = neg.reshape(NW, NCHUNK, CHUNK)
    diff, sq = _build_sc()(u3, p3, n3, user_emb, item_emb)
    total, bpr, reg = pl.pallas_call(
        _tc_body,
        out_shape=[
            jax.ShapeDtypeStruct((1, 1), jnp.float32),
            jax.ShapeDtypeStruct((1, 1), jnp.float32),
            jax.ShapeDtypeStruct((1, 1), jnp.float32),
        ],
    )(diff.reshape(128, 128), sq.reshape(4, 128))
    return total[0, 0], bpr[0, 0], reg[0, 0]


# speedup vs baseline: 1.0076x; 1.0076x over previous
import jax
import jax.numpy as jnp
from jax import lax
from jax.experimental import pallas as pl
from jax.experimental.pallas import tpu as pltpu
from jax.experimental.pallas import tpu_sc as plsc

DIM = 64
B_TOTAL = 16384
NC = 2
NS = 16
NW = NC * NS
BPW = B_TOTAL // NW
NCHUNK = 4
CHUNK = BPW // NCHUNK
REG = 0.0001

TC_BLOCK = 2048
TC_GRID = B_TOTAL // TC_BLOCK


def _sc_body(uids_hbm, pos_hbm, neg_hbm, uemb_hbm, iemb_hbm,
             uout_hbm, pout_hbm, nout_hbm,
             idx_u, idx_p, idx_n, u_v, p_v, n_v, sem, osem):
    wid = lax.axis_index("s") * NC + lax.axis_index("c")
    pltpu.sync_copy(uids_hbm.at[wid], idx_u)
    pltpu.sync_copy(pos_hbm.at[wid], idx_p)
    pltpu.sync_copy(neg_hbm.at[wid], idx_n)
    cps = []
    for j in range(NCHUNK):
        dst = pl.ds(j * CHUNK, CHUNK)
        cps.append(pltpu.async_copy(uemb_hbm.at[idx_u.at[j]], u_v.at[dst], sem))
        cps.append(pltpu.async_copy(iemb_hbm.at[idx_p.at[j]], p_v.at[dst], sem))
        cps.append(pltpu.async_copy(iemb_hbm.at[idx_n.at[j]], n_v.at[dst], sem))
    for c in cps:
        c.wait()
    out = pl.ds(wid * BPW, BPW)
    ocp = [
        pltpu.async_copy(u_v, uout_hbm.at[out], osem),
        pltpu.async_copy(p_v, pout_hbm.at[out], osem),
        pltpu.async_copy(n_v, nout_hbm.at[out], osem),
    ]
    for c in ocp:
        c.wait()


def _build_sc():
    mesh = plsc.VectorSubcoreMesh(
        core_axis_name="c", subcore_axis_name="s",
        num_cores=NC, num_subcores=NS)
    return pl.kernel(
        _sc_body,
        out_type=[
            jax.ShapeDtypeStruct((B_TOTAL, DIM), jnp.float32),
            jax.ShapeDtypeStruct((B_TOTAL, DIM), jnp.float32),
            jax.ShapeDtypeStruct((B_TOTAL, DIM), jnp.float32),
        ],
        mesh=mesh,
        compiler_params=pltpu.CompilerParams(
            needs_layout_passes=False, use_tc_tiling_on_sc=False),
        scratch_types=[
            pltpu.VMEM((NCHUNK, CHUNK), jnp.int32),
            pltpu.VMEM((NCHUNK, CHUNK), jnp.int32),
            pltpu.VMEM((NCHUNK, CHUNK), jnp.int32),
            pltpu.VMEM((BPW, DIM), jnp.float32),
            pltpu.VMEM((BPW, DIM), jnp.float32),
            pltpu.VMEM((BPW, DIM), jnp.float32),
            pltpu.SemaphoreType.DMA,
            pltpu.SemaphoreType.DMA,
        ],
    )


def _tc_body(u_ref, p_ref, n_ref, total_ref, bpr_ref, reg_ref):
    i = pl.program_id(0)
    u = u_ref[...]
    p = p_ref[...]
    n = n_ref[...]
    d = jnp.sum(u * (p - n), axis=1)
    ls = jnp.sum(jnp.log(jax.nn.sigmoid(d)))
    sq = jnp.sum(u * u) + jnp.sum(p * p) + jnp.sum(n * n)

    @pl.when(i == 0)
    def _():
        bpr_ref[...] = jnp.zeros_like(bpr_ref)
        reg_ref[...] = jnp.zeros_like(reg_ref)

    bpr_ref[...] += ls
    reg_ref[...] += sq

    @pl.when(i == TC_GRID - 1)
    def _():
        b = -bpr_ref[...] / B_TOTAL
        r = REG * (reg_ref[...] / B_TOTAL)
        bpr_ref[...] = b
        reg_ref[...] = r
        total_ref[...] = b + r


def kernel(uids, pos, neg, user_emb, item_emb):
    u3 = uids.reshape(NW, NCHUNK, CHUNK)
    p3 = pos.reshape(NW, NCHUNK, CHUNK)
    n3 = neg.reshape(NW, NCHUNK, CHUNK)
    ue, pe, ne = _build_sc()(u3, p3, n3, user_emb, item_emb)
    total, bpr, reg = pl.pallas_call(
        _tc_body,
        grid=(TC_GRID,),
        in_specs=[
            pl.BlockSpec((TC_BLOCK, DIM), lambda i: (i, 0)),
            pl.BlockSpec((TC_BLOCK, DIM), lambda i: (i, 0)),
            pl.BlockSpec((TC_BLOCK, DIM), lambda i: (i, 0)),
        ],
        out_specs=[
            pl.BlockSpec((1, 1), lambda i: (0, 0)),
            pl.BlockSpec((1, 1), lambda i: (0, 0)),
            pl.BlockSpec((1, 1), lambda i: (0, 0)),
        ],
        out_shape=[
            jax.ShapeDtypeStruct((1, 1), jnp.float32),
            jax.ShapeDtypeStruct((1, 1), jnp.float32),
            jax.ShapeDtypeStruct((1, 1), jnp.float32),
        ],
    )(ue, pe, ne)
    return total[0, 0], bpr[0, 0], reg[0, 0]
